# baseline (speedup 1.0000x reference)
import functools

import jax
import jax.numpy as jnp
from jax import lax
from jax.experimental import pallas as pl
from jax.experimental.pallas import tpu as pltpu
from jax.experimental.pallas import tpu_sc as plsc

D = 1024
FF = 4096
E = 8
K = 2
T = 2048
THRESHOLD = 0.0
REPLACEMENT = 0.0

A = T * K
BM = 256
BF = 2048
NJ = FF // BF
A_PAD = A + E * BM
NT = A_PAD // BM

NC = 2
NS = 16
LL = 16
NW = NC * NS

BG = 512


def _gating_body(x_ref, wg_ref, e2_ref, r2_ref, g2_ref, cnt_ref, carry_ref):
    c = pl.program_id(0)

    @pl.when(c == 0)
    def _():
        carry_ref[...] = jnp.zeros_like(carry_ref)

    logits = jnp.dot(x_ref[...], wg_ref[...], preferred_element_type=jnp.float32)
    iota_e = lax.broadcasted_iota(jnp.int32, (BG, E), 1)
    m1 = jnp.max(logits, axis=1, keepdims=True)
    i1 = jnp.min(jnp.where(logits == m1, iota_e, E), axis=1, keepdims=True)
    oh0 = iota_e == i1
    masked = jnp.where(oh0, -jnp.inf, logits)
    m2 = jnp.max(masked, axis=1, keepdims=True)
    i2 = jnp.min(jnp.where(masked == m2, iota_e, E), axis=1, keepdims=True)
    oh1 = iota_e == i2

    w = jnp.exp(m2 - m1)
    g1 = 1.0 / (1.0 + w)
    g2 = w / (1.0 + w)

    oh0f = oh0.astype(jnp.float32)
    oh1f = oh1.astype(jnp.float32)
    s = oh0f + oh1f
    it_r = lax.broadcasted_iota(jnp.int32, (BG, BG), 0)
    it_c = lax.broadcasted_iota(jnp.int32, (BG, BG), 1)
    tri = (it_r > it_c).astype(jnp.float32)
    cum = jnp.dot(tri, s, preferred_element_type=jnp.float32) + carry_ref[...]
    r0 = jnp.sum(cum * oh0f, axis=1, keepdims=True)
    r1 = jnp.sum((cum + oh0f) * oh1f, axis=1, keepdims=True)
    carry_new = carry_ref[...] + jnp.sum(s, axis=0, keepdims=True)
    carry_ref[...] = carry_new

    e2_ref[...] = jnp.concatenate([i1, i2], axis=1)
    r2_ref[...] = jnp.concatenate([r0, r1], axis=1).astype(jnp.int32)
    g2_ref[...] = jnp.concatenate([g1, g2], axis=1)
    cnt_ref[...] = carry_new.astype(jnp.int32)


def _gating(x, w_gate):
    return pl.pallas_call(
        _gating_body,
        grid=(T // BG,),
        in_specs=[
            pl.BlockSpec((BG, D), lambda c: (c, 0)),
            pl.BlockSpec((D, E), lambda c: (0, 0)),
        ],
        out_specs=[
            pl.BlockSpec((BG, K), lambda c: (c, 0)),
            pl.BlockSpec((BG, K), lambda c: (c, 0)),
            pl.BlockSpec((BG, K), lambda c: (c, 0)),
            pl.BlockSpec((1, E), lambda c: (0, 0)),
        ],
        out_shape=[
            jax.ShapeDtypeStruct((T, K), jnp.int32),
            jax.ShapeDtypeStruct((T, K), jnp.int32),
            jax.ShapeDtypeStruct((T, K), jnp.float32),
            jax.ShapeDtypeStruct((1, E), jnp.int32),
        ],
        scratch_shapes=[pltpu.VMEM((1, E), jnp.float32)],
    )(x, w_gate)


_PW = A // NW
_KCH = 32
_NCH = _PW // _KCH

@functools.cache
def _sc_mesh():
    return plsc.VectorSubcoreMesh(
        core_axis_name="c", subcore_axis_name="s",
        num_cores=NC, num_subcores=NS)


@functools.cache
def _dispatch_kernel():
    @functools.partial(
        pl.kernel,
        mesh=_sc_mesh(),
        out_type=[
            jax.ShapeDtypeStruct((A_PAD, D), jnp.float32),
            jax.ShapeDtypeStruct((A,), jnp.int32),
        ],
        scratch_types=[
            pltpu.VMEM((_KCH,), jnp.int32),
            pltpu.VMEM((_KCH,), jnp.int32),
            pltpu.VMEM((E,), jnp.int32),
            pltpu.VMEM((1, _KCH), jnp.int32),
            pltpu.VMEM((1, _KCH), jnp.int32),
            pltpu.VMEM((_KCH, D), jnp.float32),
            pltpu.SemaphoreType.DMA,
            pltpu.SemaphoreType.DMA,
        ],
        compiler_params=pltpu.CompilerParams(needs_layout_passes=False),
    )
    def _dispatch_k(x_hbm, ef_hbm, rf_hbm, off_hbm, xs_hbm, dest_hbm,
                    ef_v, rf_v, off_v, tok_v, dst_v, rows_v, sem_g, sem_s):
        wid = lax.axis_index("s") * NC + lax.axis_index("c")
        pltpu.sync_copy(off_hbm, off_v)
        for ci in range(_NCH):
            base = wid * _PW + ci * _KCH
            pltpu.sync_copy(ef_hbm.at[pl.ds(base, _KCH)], ef_v)
            pltpu.sync_copy(rf_hbm.at[pl.ds(base, _KCH)], rf_v)
            for s in range(_KCH // LL):
                e16 = ef_v[pl.ds(s * LL, LL)]
                off16 = plsc.load_gather(off_v, [e16])
                dst_v[0, pl.ds(s * LL, LL)] = off16 + rf_v[pl.ds(s * LL, LL)]
                t16 = (base + s * LL + lax.iota(jnp.int32, LL)) // K
                tok_v[0, pl.ds(s * LL, LL)] = t16
            pltpu.async_copy(x_hbm.at[tok_v.at[0]], rows_v, sem_g).wait()
            pltpu.async_copy(rows_v, xs_hbm.at[dst_v.at[0]], sem_s).wait()
            pltpu.sync_copy(dst_v.at[0], dest_hbm.at[pl.ds(base, _KCH)])

    return _dispatch_k


def _ffn_body(eot_ref, xs_ref, w1_ref, b1_ref, w2_ref, b2_ref, ys_ref, acc_ref):
    del eot_ref
    j = pl.program_id(1)
    xb = xs_ref[...].astype(jnp.bfloat16)
    h = jnp.dot(xb, w1_ref[0], preferred_element_type=jnp.float32)
    h = h + b1_ref[0]
    a = jnp.maximum(h, 0.0)
    a = jnp.where(a > THRESHOLD, a, REPLACEMENT)
    p = jnp.dot(a.astype(jnp.bfloat16), w2_ref[0],
                preferred_element_type=jnp.float32)

    @pl.when(j == 0)
    def _():
        acc_ref[...] = p

    @pl.when(j > 0)
    def _():
        acc_ref[...] += p

    @pl.when(j == NJ - 1)
    def _():
        ys_ref[...] = acc_ref[...] + b2_ref[0]


def _ffn(eot, xs, W1, b1, W2, b2):
    grid_spec = pltpu.PrefetchScalarGridSpec(
        num_scalar_prefetch=1,
        grid=(NT, NJ),
        in_specs=[
            pl.BlockSpec((BM, D), lambda i, j, eot: (i, 0)),
            pl.BlockSpec((1, D, BF), lambda i, j, eot: (eot[i], 0, j)),
            pl.BlockSpec((1, 1, BF), lambda i, j, eot: (eot[i], 0, j)),
            pl.BlockSpec((1, BF, D), lambda i, j, eot: (eot[i], j, 0)),
            pl.BlockSpec((1, 1, D), lambda i, j, eot: (eot[i], 0, 0)),
        ],
        out_specs=pl.BlockSpec((BM, D), lambda i, j, eot: (i, 0)),
        scratch_shapes=[pltpu.VMEM((BM, D), jnp.float32)],
    )
    return pl.pallas_call(
        _ffn_body,
        grid_spec=grid_spec,
        out_shape=jax.ShapeDtypeStruct((A_PAD, D), jnp.float32),
        compiler_params=pltpu.CompilerParams(
            dimension_semantics=("arbitrary", "arbitrary")),
    )(eot, xs, W1.astype(jnp.bfloat16), b1.reshape(E, 1, FF),
      W2.astype(jnp.bfloat16), b2.reshape(E, 1, D))


_TPW = T // NW
_TCH = 32
_NCH2 = _TPW // _TCH


@functools.cache
def _combine_kernel():
    @functools.partial(
        pl.kernel,
        mesh=_sc_mesh(),
        out_type=jax.ShapeDtypeStruct((T, D), jnp.float32),
        scratch_types=[
            pltpu.VMEM((1, K * _TCH), jnp.int32),
            pltpu.VMEM((_TCH, K), jnp.float32),
            pltpu.VMEM((K * _TCH, D), jnp.float32),
            pltpu.VMEM((_TCH, D), jnp.float32),
            pltpu.SemaphoreType.DMA,
        ],
        compiler_params=pltpu.CompilerParams(needs_layout_passes=False),
    )
    def _combine_k(ys_hbm, dest_hbm, g2_hbm, out_hbm, d_v, g_v, rows_v, o_v, sem):
        wid = lax.axis_index("s") * NC + lax.axis_index("c")
        for ci in range(_NCH2):
            t0 = wid * _TPW + ci * _TCH
            pltpu.sync_copy(dest_hbm.at[pl.ds(K * t0, K * _TCH)], d_v.at[0])
            pltpu.sync_copy(g2_hbm.at[pl.ds(t0, _TCH)], g_v)
            pltpu.async_copy(ys_hbm.at[d_v.at[0]], rows_v, sem).wait()

            def body(t, _):
                tt = jnp.full((LL,), t, jnp.int32)
                g0 = plsc.load_gather(g_v, [tt, jnp.zeros((LL,), jnp.int32)])
                g1 = plsc.load_gather(g_v, [tt, jnp.ones((LL,), jnp.int32)])
                for dc in range(D // LL):
                    a = rows_v[2 * t, pl.ds(dc * LL, LL)]
                    b = rows_v[2 * t + 1, pl.ds(dc * LL, LL)]
                    o_v[t, pl.ds(dc * LL, LL)] = a * g0 + b * g1
                return 0

            lax.fori_loop(0, _TCH, body, 0)
            pltpu.sync_copy(o_v, out_hbm.at[pl.ds(t0, _TCH)])

    return _combine_k


def kernel(h, w_gate, W1, b1, W2, b2):
    Bb, Ss, Dd = h.shape
    x = h.reshape(T, D)

    e2, r2, g2, cnt = _gating(x, w_gate)

    counts = cnt[0]
    padded = ((counts + BM - 1) // BM) * BM
    ends = jnp.cumsum(padded)
    offsets = (ends - padded).astype(jnp.int32)
    tb = jnp.arange(NT, dtype=jnp.int32) * BM
    eot = jnp.minimum(
        jnp.sum((tb[:, None] >= ends[None, :]).astype(jnp.int32), axis=1),
        E - 1).astype(jnp.int32)

    xs, dest = _dispatch_kernel()(x, e2.reshape(A), r2.reshape(A), offsets)
    ys = _ffn(eot, xs, W1, b1, W2, b2)
    out = _combine_kernel()(ys, dest, g2)
    return out.reshape(Bb, Ss, Dd)


# speedup vs baseline: 1.1446x; 1.1446x over previous
import functools

import jax
import jax.numpy as jnp
from jax import lax
from jax.experimental import pallas as pl
from jax.experimental.pallas import tpu as pltpu
from jax.experimental.pallas import tpu_sc as plsc

D = 1024
FF = 4096
E = 8
K = 2
T = 2048
THRESHOLD = 0.0
REPLACEMENT = 0.0

A = T * K
BM = 256
BF = 1024
NJ = FF // BF
A_PAD = A + E * BM
NT = A_PAD // BM

NC = 2
NS = 16
LL = 16
NW = NC * NS

BG = 512


def _gating_body(x_ref, wg_ref, e2_ref, r2_ref, g2_ref, cnt_ref, carry_ref):
    c = pl.program_id(0)

    @pl.when(c == 0)
    def _():
        carry_ref[...] = jnp.zeros_like(carry_ref)

    logits = jnp.dot(x_ref[...], wg_ref[...], preferred_element_type=jnp.float32)
    iota_e = lax.broadcasted_iota(jnp.int32, (BG, E), 1)
    m1 = jnp.max(logits, axis=1, keepdims=True)
    i1 = jnp.min(jnp.where(logits == m1, iota_e, E), axis=1, keepdims=True)
    oh0 = iota_e == i1
    masked = jnp.where(oh0, -jnp.inf, logits)
    m2 = jnp.max(masked, axis=1, keepdims=True)
    i2 = jnp.min(jnp.where(masked == m2, iota_e, E), axis=1, keepdims=True)
    oh1 = iota_e == i2

    w = jnp.exp(m2 - m1)
    g1 = 1.0 / (1.0 + w)
    g2 = w / (1.0 + w)

    oh0f = oh0.astype(jnp.float32)
    oh1f = oh1.astype(jnp.float32)
    s = oh0f + oh1f
    it_r = lax.broadcasted_iota(jnp.int32, (BG, BG), 0)
    it_c = lax.broadcasted_iota(jnp.int32, (BG, BG), 1)
    tri = (it_r > it_c).astype(jnp.float32)
    cum = jnp.dot(tri, s, preferred_element_type=jnp.float32) + carry_ref[...]
    r0 = jnp.sum(cum * oh0f, axis=1, keepdims=True)
    r1 = jnp.sum((cum + oh0f) * oh1f, axis=1, keepdims=True)
    carry_new = carry_ref[...] + jnp.sum(s, axis=0, keepdims=True)
    carry_ref[...] = carry_new

    e2_ref[...] = jnp.concatenate([i1, i2], axis=1)
    r2_ref[...] = jnp.concatenate([r0, r1], axis=1).astype(jnp.int32)
    g2_ref[...] = jnp.concatenate([g1, g2], axis=1)
    cnt_ref[...] = carry_new.astype(jnp.int32)


def _gating(x, w_gate):
    return pl.pallas_call(
        _gating_body,
        grid=(T // BG,),
        in_specs=[
            pl.BlockSpec((BG, D), lambda c: (c, 0)),
            pl.BlockSpec((D, E), lambda c: (0, 0)),
        ],
        out_specs=[
            pl.BlockSpec((BG, K), lambda c: (c, 0)),
            pl.BlockSpec((BG, K), lambda c: (c, 0)),
            pl.BlockSpec((BG, K), lambda c: (c, 0)),
            pl.BlockSpec((1, E), lambda c: (0, 0)),
        ],
        out_shape=[
            jax.ShapeDtypeStruct((T, K), jnp.int32),
            jax.ShapeDtypeStruct((T, K), jnp.int32),
            jax.ShapeDtypeStruct((T, K), jnp.float32),
            jax.ShapeDtypeStruct((1, E), jnp.int32),
        ],
        scratch_shapes=[pltpu.VMEM((1, E), jnp.float32)],
    )(x, w_gate)


_PW = A // NW
_KCH = 32
_NCH = _PW // _KCH

@functools.cache
def _sc_mesh():
    return plsc.VectorSubcoreMesh(
        core_axis_name="c", subcore_axis_name="s",
        num_cores=NC, num_subcores=NS)


@functools.cache
def _dispatch_kernel():
    @functools.partial(
        pl.kernel,
        mesh=_sc_mesh(),
        out_type=[
            jax.ShapeDtypeStruct((A_PAD, D), jnp.float32),
            jax.ShapeDtypeStruct((A,), jnp.int32),
        ],
        scratch_types=[
            pltpu.VMEM((_KCH,), jnp.int32),
            pltpu.VMEM((_KCH,), jnp.int32),
            pltpu.VMEM((E,), jnp.int32),
            pltpu.VMEM((1, _KCH), jnp.int32),
            pltpu.VMEM((1, _KCH), jnp.int32),
            pltpu.VMEM((_KCH, D), jnp.float32),
            pltpu.SemaphoreType.DMA,
            pltpu.SemaphoreType.DMA,
        ],
        compiler_params=pltpu.CompilerParams(needs_layout_passes=False),
    )
    def _dispatch_k(x_hbm, ef_hbm, rf_hbm, off_hbm, xs_hbm, dest_hbm,
                    ef_v, rf_v, off_v, tok_v, dst_v, rows_v, sem_g, sem_s):
        wid = lax.axis_index("s") * NC + lax.axis_index("c")
        pltpu.sync_copy(off_hbm, off_v)
        for ci in range(_NCH):
            base = wid * _PW + ci * _KCH
            pltpu.sync_copy(ef_hbm.at[pl.ds(base, _KCH)], ef_v)
            pltpu.sync_copy(rf_hbm.at[pl.ds(base, _KCH)], rf_v)
            for s in range(_KCH // LL):
                e16 = ef_v[pl.ds(s * LL, LL)]
                off16 = plsc.load_gather(off_v, [e16])
                dst_v[0, pl.ds(s * LL, LL)] = off16 + rf_v[pl.ds(s * LL, LL)]
                t16 = (base + s * LL + lax.iota(jnp.int32, LL)) // K
                tok_v[0, pl.ds(s * LL, LL)] = t16
            pltpu.async_copy(x_hbm.at[tok_v.at[0]], rows_v, sem_g).wait()
            pltpu.async_copy(rows_v, xs_hbm.at[dst_v.at[0]], sem_s).wait()
            pltpu.sync_copy(dst_v.at[0], dest_hbm.at[pl.ds(base, _KCH)])

    return _dispatch_k


def _ffn_body(eot_ref, xs_ref, w1_ref, b1_ref, w2_ref, b2_ref, ys_ref, acc_ref):
    del eot_ref
    j = pl.program_id(0)
    i = pl.program_id(1)
    h = jnp.dot(xs_ref[...], w1_ref[0], preferred_element_type=jnp.float32)
    h = h + b1_ref[0]
    a = jnp.maximum(h, 0.0)
    a = jnp.where(a > THRESHOLD, a, REPLACEMENT)
    p = jnp.dot(a, w2_ref[0], preferred_element_type=jnp.float32)
    row = pl.ds(i * BM, BM)

    @pl.when(j == 0)
    def _():
        acc_ref[row, :] = p

    @pl.when(j > 0)
    def _():
        acc_ref[row, :] += p

    @pl.when(j == NJ - 1)
    def _():
        ys_ref[...] = acc_ref[row, :] + b2_ref[0]


def _ffn(eot, xs, W1, b1, W2, b2):
    grid_spec = pltpu.PrefetchScalarGridSpec(
        num_scalar_prefetch=1,
        grid=(NJ, NT),
        in_specs=[
            pl.BlockSpec((BM, D), lambda j, i, eot: (i, 0)),
            pl.BlockSpec((1, D, BF), lambda j, i, eot: (eot[i], 0, j)),
            pl.BlockSpec((1, 1, BF), lambda j, i, eot: (eot[i], 0, j)),
            pl.BlockSpec((1, BF, D), lambda j, i, eot: (eot[i], j, 0)),
            pl.BlockSpec((1, 1, D), lambda j, i, eot: (eot[i], 0, 0)),
        ],
        out_specs=pl.BlockSpec(
            (BM, D), lambda j, i, eot: (jnp.where(j == NJ - 1, i, 0), 0)),
        scratch_shapes=[pltpu.VMEM((A_PAD, D), jnp.float32)],
    )
    return pl.pallas_call(
        _ffn_body,
        grid_spec=grid_spec,
        out_shape=jax.ShapeDtypeStruct((A_PAD, D), jnp.float32),
        compiler_params=pltpu.CompilerParams(
            dimension_semantics=("arbitrary", "arbitrary")),
    )(eot, xs, W1, b1.reshape(E, 1, FF), W2, b2.reshape(E, 1, D))


_TPW = T // NW
_TCH = 32
_NCH2 = _TPW // _TCH


@functools.cache
def _combine_kernel():
    @functools.partial(
        pl.kernel,
        mesh=_sc_mesh(),
        out_type=jax.ShapeDtypeStruct((T, D), jnp.float32),
        scratch_types=[
            pltpu.VMEM((1, K * _TCH), jnp.int32),
            pltpu.VMEM((_TCH, K), jnp.float32),
            pltpu.VMEM((K * _TCH, D), jnp.float32),
            pltpu.VMEM((_TCH, D), jnp.float32),
            pltpu.SemaphoreType.DMA,
        ],
        compiler_params=pltpu.CompilerParams(needs_layout_passes=False),
    )
    def _combine_k(ys_hbm, dest_hbm, g2_hbm, out_hbm, d_v, g_v, rows_v, o_v, sem):
        wid = lax.axis_index("s") * NC + lax.axis_index("c")
        for ci in range(_NCH2):
            t0 = wid * _TPW + ci * _TCH
            pltpu.sync_copy(dest_hbm.at[pl.ds(K * t0, K * _TCH)], d_v.at[0])
            pltpu.sync_copy(g2_hbm.at[pl.ds(t0, _TCH)], g_v)
            pltpu.async_copy(ys_hbm.at[d_v.at[0]], rows_v, sem).wait()

            def body(t, _):
                tt = jnp.full((LL,), t, jnp.int32)
                g0 = plsc.load_gather(g_v, [tt, jnp.zeros((LL,), jnp.int32)])
                g1 = plsc.load_gather(g_v, [tt, jnp.ones((LL,), jnp.int32)])
                for dc in range(D // LL):
                    a = rows_v[2 * t, pl.ds(dc * LL, LL)]
                    b = rows_v[2 * t + 1, pl.ds(dc * LL, LL)]
                    o_v[t, pl.ds(dc * LL, LL)] = a * g0 + b * g1
                return 0

            lax.fori_loop(0, _TCH, body, 0)
            pltpu.sync_copy(o_v, out_hbm.at[pl.ds(t0, _TCH)])

    return _combine_k


def kernel(h, w_gate, W1, b1, W2, b2):
    Bb, Ss, Dd = h.shape
    x = h.reshape(T, D)

    e2, r2, g2, cnt = _gating(x, w_gate)

    counts = cnt[0]
    padded = ((counts + BM - 1) // BM) * BM
    ends = jnp.cumsum(padded)
    offsets = (ends - padded).astype(jnp.int32)
    tb = jnp.arange(NT, dtype=jnp.int32) * BM
    eot = jnp.minimum(
        jnp.sum((tb[:, None] >= ends[None, :]).astype(jnp.int32), axis=1),
        E - 1).astype(jnp.int32)

    xs, dest = _dispatch_kernel()(x, e2.reshape(A), r2.reshape(A), offsets)
    ys = _ffn(eot, xs, W1, b1, W2, b2)
    out = _combine_kernel()(ys, dest, g2)
    return out.reshape(Bb, Ss, Dd)


# speedup vs baseline: 1.3375x; 1.1686x over previous
import functools

import jax
import jax.numpy as jnp
from jax import lax
from jax.experimental import pallas as pl
from jax.experimental.pallas import tpu as pltpu
from jax.experimental.pallas import tpu_sc as plsc

D = 1024
FF = 4096
E = 8
K = 2
T = 2048
THRESHOLD = 0.0
REPLACEMENT = 0.0

A = T * K
BM = 256
BF = 1024
NJ = FF // BF
A_PAD = A + E * BM
NT = A_PAD // BM

NC = 2
NS = 16
LL = 16
NW = NC * NS

BG = 512


def _gating_body(x_ref, wg_ref, e2_ref, r2_ref, g2_ref, cnt_ref, carry_ref):
    c = pl.program_id(0)

    @pl.when(c == 0)
    def _():
        carry_ref[...] = jnp.zeros_like(carry_ref)

    logits = jnp.dot(x_ref[...], wg_ref[...], preferred_element_type=jnp.float32)
    iota_e = lax.broadcasted_iota(jnp.int32, (BG, E), 1)
    m1 = jnp.max(logits, axis=1, keepdims=True)
    i1 = jnp.min(jnp.where(logits == m1, iota_e, E), axis=1, keepdims=True)
    oh0 = iota_e == i1
    masked = jnp.where(oh0, -jnp.inf, logits)
    m2 = jnp.max(masked, axis=1, keepdims=True)
    i2 = jnp.min(jnp.where(masked == m2, iota_e, E), axis=1, keepdims=True)
    oh1 = iota_e == i2

    w = jnp.exp(m2 - m1)
    g1 = 1.0 / (1.0 + w)
    g2 = w / (1.0 + w)

    oh0f = oh0.astype(jnp.float32)
    oh1f = oh1.astype(jnp.float32)
    s = oh0f + oh1f
    it_r = lax.broadcasted_iota(jnp.int32, (BG, BG), 0)
    it_c = lax.broadcasted_iota(jnp.int32, (BG, BG), 1)
    tri = (it_r > it_c).astype(jnp.float32)
    cum = jnp.dot(tri, s, preferred_element_type=jnp.float32) + carry_ref[...]
    r0 = jnp.sum(cum * oh0f, axis=1, keepdims=True)
    r1 = jnp.sum((cum + oh0f) * oh1f, axis=1, keepdims=True)
    carry_new = carry_ref[...] + jnp.sum(s, axis=0, keepdims=True)
    carry_ref[...] = carry_new

    e2_ref[...] = jnp.concatenate([i1, i2], axis=1)
    r2_ref[...] = jnp.concatenate([r0, r1], axis=1).astype(jnp.int32)
    g2_ref[...] = jnp.concatenate([g1, g2], axis=1)
    cnt_ref[...] = carry_new.astype(jnp.int32)


def _gating(x, w_gate):
    return pl.pallas_call(
        _gating_body,
        grid=(T // BG,),
        in_specs=[
            pl.BlockSpec((BG, D), lambda c: (c, 0)),
            pl.BlockSpec((D, E), lambda c: (0, 0)),
        ],
        out_specs=[
            pl.BlockSpec((BG, K), lambda c: (c, 0)),
            pl.BlockSpec((BG, K), lambda c: (c, 0)),
            pl.BlockSpec((BG, K), lambda c: (c, 0)),
            pl.BlockSpec((1, E), lambda c: (0, 0)),
        ],
        out_shape=[
            jax.ShapeDtypeStruct((T, K), jnp.int32),
            jax.ShapeDtypeStruct((T, K), jnp.int32),
            jax.ShapeDtypeStruct((T, K), jnp.float32),
            jax.ShapeDtypeStruct((1, E), jnp.int32),
        ],
        scratch_shapes=[pltpu.VMEM((1, E), jnp.float32)],
    )(x, w_gate)


_PW = A // NW
_KCH = 32
_NCH = _PW // _KCH

@functools.cache
def _sc_mesh():
    return plsc.VectorSubcoreMesh(
        core_axis_name="c", subcore_axis_name="s",
        num_cores=NC, num_subcores=NS)


@functools.cache
def _dispatch_kernel():
    @functools.partial(
        pl.kernel,
        mesh=_sc_mesh(),
        out_type=[
            jax.ShapeDtypeStruct((A_PAD, D), jnp.float32),
            jax.ShapeDtypeStruct((A,), jnp.int32),
        ],
        scratch_types=[
            pltpu.VMEM((_KCH,), jnp.int32),
            pltpu.VMEM((_KCH,), jnp.int32),
            pltpu.VMEM((E,), jnp.int32),
            pltpu.VMEM((1, _KCH), jnp.int32),
            pltpu.VMEM((1, _KCH), jnp.int32),
            pltpu.VMEM((_KCH, D), jnp.float32),
            pltpu.SemaphoreType.DMA,
            pltpu.SemaphoreType.DMA,
        ],
        compiler_params=pltpu.CompilerParams(needs_layout_passes=False),
    )
    def _dispatch_k(x_hbm, ef_hbm, rf_hbm, off_hbm, xs_hbm, dest_hbm,
                    ef_v, rf_v, off_v, tok_v, dst_v, rows_v, sem_g, sem_s):
        wid = lax.axis_index("s") * NC + lax.axis_index("c")
        pltpu.sync_copy(off_hbm, off_v)
        for ci in range(_NCH):
            base = wid * _PW + ci * _KCH
            pltpu.sync_copy(ef_hbm.at[pl.ds(base, _KCH)], ef_v)
            pltpu.sync_copy(rf_hbm.at[pl.ds(base, _KCH)], rf_v)
            for s in range(_KCH // LL):
                e16 = ef_v[pl.ds(s * LL, LL)]
                off16 = plsc.load_gather(off_v, [e16])
                dst_v[0, pl.ds(s * LL, LL)] = off16 + rf_v[pl.ds(s * LL, LL)]
                t16 = (base + s * LL + lax.iota(jnp.int32, LL)) // K
                tok_v[0, pl.ds(s * LL, LL)] = t16
            pltpu.async_copy(x_hbm.at[tok_v.at[0]], rows_v, sem_g).wait()
            pltpu.async_copy(rows_v, xs_hbm.at[dst_v.at[0]], sem_s).wait()
            pltpu.sync_copy(dst_v.at[0], dest_hbm.at[pl.ds(base, _KCH)])

    return _dispatch_k


def _start_w(w1_ref, w2_ref, e, jj, slot, w1b_ref, w2b_ref, sem_ref):
    pltpu.make_async_copy(
        w1_ref.at[e, :, pl.ds(jj * BF, BF)], w1b_ref.at[slot], sem_ref.at[slot]
    ).start()
    pltpu.make_async_copy(
        w2_ref.at[e, pl.ds(jj * BF, BF), :], w2b_ref.at[slot], sem_ref.at[slot]
    ).start()


def _ffn_body(p_ref, xs_ref, w1_ref, b1_ref, w2_ref, b2_ref, ys_ref,
              w1b_ref, w2b_ref, acc_ref, sem_ref):
    j = pl.program_id(0)
    i = pl.program_id(1)
    e = p_ref[0, i]
    first = p_ref[1, i] == 1
    slot = lax.rem(p_ref[2, i] + j * p_ref[5, 0], 2)

    @pl.when((j == 0) & (i == 0))
    def _():
        _start_w(w1_ref, w2_ref, e, 0, slot, w1b_ref, w2b_ref, sem_ref)

    @pl.when(first)
    def _():
        pltpu.make_async_copy(
            w1_ref.at[0, :, pl.ds(0, BF)], w1b_ref.at[slot], sem_ref.at[slot]
        ).wait()
        pltpu.make_async_copy(
            w2_ref.at[0, pl.ds(0, BF), :], w2b_ref.at[slot], sem_ref.at[slot]
        ).wait()
        ne = p_ref[3, i]
        lr = p_ref[4, i]
        jn = j + lr

        @pl.when(jn < NJ)
        def _():
            _start_w(w1_ref, w2_ref, ne, jn, 1 - slot, w1b_ref, w2b_ref,
                     sem_ref)

    h = jnp.dot(xs_ref[...], w1b_ref[slot], preferred_element_type=jnp.float32)
    h = h + b1_ref[0]
    a = jnp.maximum(h, 0.0)
    a = jnp.where(a > THRESHOLD, a, REPLACEMENT)
    p = jnp.dot(a, w2b_ref[slot], preferred_element_type=jnp.float32)
    row = pl.ds(i * BM, BM)

    @pl.when(j == 0)
    def _():
        acc_ref[row, :] = p

    @pl.when(j > 0)
    def _():
        acc_ref[row, :] += p

    @pl.when(j == NJ - 1)
    def _():
        ys_ref[...] = acc_ref[row, :] + b2_ref[0]


def _ffn(pref, xs, W1, b1, W2, b2):
    grid_spec = pltpu.PrefetchScalarGridSpec(
        num_scalar_prefetch=1,
        grid=(NJ, NT),
        in_specs=[
            pl.BlockSpec((BM, D), lambda j, i, p: (i, 0)),
            pl.BlockSpec(memory_space=pltpu.MemorySpace.HBM),
            pl.BlockSpec((1, 1, BF), lambda j, i, p: (p[0, i], 0, j)),
            pl.BlockSpec(memory_space=pltpu.MemorySpace.HBM),
            pl.BlockSpec((1, 1, D), lambda j, i, p: (p[0, i], 0, 0)),
        ],
        out_specs=pl.BlockSpec(
            (BM, D), lambda j, i, p: (jnp.where(j == NJ - 1, i, 0), 0)),
        scratch_shapes=[
            pltpu.VMEM((2, D, BF), jnp.float32),
            pltpu.VMEM((2, BF, D), jnp.float32),
            pltpu.VMEM((A_PAD, D), jnp.float32),
            pltpu.SemaphoreType.DMA((2,)),
        ],
    )
    return pl.pallas_call(
        _ffn_body,
        grid_spec=grid_spec,
        out_shape=jax.ShapeDtypeStruct((A_PAD, D), jnp.float32),
        compiler_params=pltpu.CompilerParams(
            dimension_semantics=("arbitrary", "arbitrary")),
    )(pref, xs, W1, b1.reshape(E, 1, FF), W2, b2.reshape(E, 1, D))


_TPW = T // NW
_TCH = 32
_NCH2 = _TPW // _TCH


@functools.cache
def _combine_kernel():
    @functools.partial(
        pl.kernel,
        mesh=_sc_mesh(),
        out_type=jax.ShapeDtypeStruct((T, D), jnp.float32),
        scratch_types=[
            pltpu.VMEM((1, K * _TCH), jnp.int32),
            pltpu.VMEM((_TCH, K), jnp.float32),
            pltpu.VMEM((K * _TCH, D), jnp.float32),
            pltpu.VMEM((_TCH, D), jnp.float32),
            pltpu.SemaphoreType.DMA,
        ],
        compiler_params=pltpu.CompilerParams(needs_layout_passes=False),
    )
    def _combine_k(ys_hbm, dest_hbm, g2_hbm, out_hbm, d_v, g_v, rows_v, o_v, sem):
        wid = lax.axis_index("s") * NC + lax.axis_index("c")
        for ci in range(_NCH2):
            t0 = wid * _TPW + ci * _TCH
            pltpu.sync_copy(dest_hbm.at[pl.ds(K * t0, K * _TCH)], d_v.at[0])
            pltpu.sync_copy(g2_hbm.at[pl.ds(t0, _TCH)], g_v)
            pltpu.async_copy(ys_hbm.at[d_v.at[0]], rows_v, sem).wait()

            def body(t, _):
                tt = jnp.full((LL,), t, jnp.int32)
                g0 = plsc.load_gather(g_v, [tt, jnp.zeros((LL,), jnp.int32)])
                g1 = plsc.load_gather(g_v, [tt, jnp.ones((LL,), jnp.int32)])
                for dc in range(D // LL):
                    a = rows_v[2 * t, pl.ds(dc * LL, LL)]
                    b = rows_v[2 * t + 1, pl.ds(dc * LL, LL)]
                    o_v[t, pl.ds(dc * LL, LL)] = a * g0 + b * g1
                return 0

            lax.fori_loop(0, _TCH, body, 0)
            pltpu.sync_copy(o_v, out_hbm.at[pl.ds(t0, _TCH)])

    return _combine_k


def kernel(h, w_gate, W1, b1, W2, b2):
    Bb, Ss, Dd = h.shape
    x = h.reshape(T, D)

    e2, r2, g2, cnt = _gating(x, w_gate)

    counts = cnt[0]
    padded = ((counts + BM - 1) // BM) * BM
    ends = jnp.cumsum(padded)
    offsets = (ends - padded).astype(jnp.int32)
    tb = jnp.arange(NT, dtype=jnp.int32) * BM
    eot = jnp.minimum(
        jnp.sum((tb[:, None] >= ends[None, :]).astype(jnp.int32), axis=1),
        E - 1).astype(jnp.int32)
    first = jnp.concatenate([jnp.ones((1,), jnp.int32),
                             (eot[1:] != eot[:-1]).astype(jnp.int32)])
    runidx = jnp.cumsum(first) - 1
    nruns = runidx[-1] + 1
    nxt = jnp.searchsorted(runidx, runidx + 1)
    nexte = eot[jnp.where(nxt >= NT, 0, nxt)]
    lastrun = (runidx == nruns - 1).astype(jnp.int32)
    rpar = jnp.full((NT,), nruns % 2, jnp.int32)
    pref = jnp.stack([eot, first, runidx % 2, nexte, lastrun, rpar]
                     ).astype(jnp.int32)

    xs, dest = _dispatch_kernel()(x, e2.reshape(A), r2.reshape(A), offsets)
    ys = _ffn(pref, xs, W1, b1, W2, b2)
    out = _combine_kernel()(ys, dest, g2)
    return out.reshape(Bb, Ss, Dd)


# speedup vs baseline: 1.4601x; 1.0917x over previous
import functools

import jax
import jax.numpy as jnp
from jax import lax
from jax.experimental import pallas as pl
from jax.experimental.pallas import tpu as pltpu
from jax.experimental.pallas import tpu_sc as plsc

D = 1024
FF = 4096
E = 8
K = 2
T = 2048
THRESHOLD = 0.0
REPLACEMENT = 0.0

A = T * K
BM = 256
BF = 1024
NJ = FF // BF
A_PAD = A + E * BM
NT = A_PAD // BM

NC = 2
NS = 16
LL = 16
NW = NC * NS

BG = 512


def _gating_body(x_ref, wg_ref, e2_ref, r2_ref, g2_ref, cnt_ref, carry_ref):
    c = pl.program_id(0)

    @pl.when(c == 0)
    def _():
        carry_ref[...] = jnp.zeros_like(carry_ref)

    logits = jnp.dot(x_ref[...], wg_ref[...], preferred_element_type=jnp.float32)
    iota_e = lax.broadcasted_iota(jnp.int32, (BG, E), 1)
    m1 = jnp.max(logits, axis=1, keepdims=True)
    i1 = jnp.min(jnp.where(logits == m1, iota_e, E), axis=1, keepdims=True)
    oh0 = iota_e == i1
    masked = jnp.where(oh0, -jnp.inf, logits)
    m2 = jnp.max(masked, axis=1, keepdims=True)
    i2 = jnp.min(jnp.where(masked == m2, iota_e, E), axis=1, keepdims=True)
    oh1 = iota_e == i2

    w = jnp.exp(m2 - m1)
    g1 = 1.0 / (1.0 + w)
    g2 = w / (1.0 + w)

    oh0f = oh0.astype(jnp.float32)
    oh1f = oh1.astype(jnp.float32)
    s = oh0f + oh1f
    it_r = lax.broadcasted_iota(jnp.int32, (BG, BG), 0)
    it_c = lax.broadcasted_iota(jnp.int32, (BG, BG), 1)
    tri = (it_r > it_c).astype(jnp.float32)
    cum = jnp.dot(tri, s, preferred_element_type=jnp.float32) + carry_ref[...]
    r0 = jnp.sum(cum * oh0f, axis=1, keepdims=True)
    r1 = jnp.sum((cum + oh0f) * oh1f, axis=1, keepdims=True)
    carry_new = carry_ref[...] + jnp.sum(s, axis=0, keepdims=True)
    carry_ref[...] = carry_new

    e2_ref[...] = jnp.concatenate([i1, i2], axis=1)
    r2_ref[...] = jnp.concatenate([r0, r1], axis=1).astype(jnp.int32)
    g2_ref[...] = jnp.concatenate([g1, g2], axis=1)
    cnt_ref[...] = carry_new.astype(jnp.int32)


def _gating(x, w_gate):
    return pl.pallas_call(
        _gating_body,
        grid=(T // BG,),
        in_specs=[
            pl.BlockSpec((BG, D), lambda c: (c, 0)),
            pl.BlockSpec((D, E), lambda c: (0, 0)),
        ],
        out_specs=[
            pl.BlockSpec((BG, K), lambda c: (c, 0)),
            pl.BlockSpec((BG, K), lambda c: (c, 0)),
            pl.BlockSpec((BG, K), lambda c: (c, 0)),
            pl.BlockSpec((1, E), lambda c: (0, 0)),
        ],
        out_shape=[
            jax.ShapeDtypeStruct((T, K), jnp.int32),
            jax.ShapeDtypeStruct((T, K), jnp.int32),
            jax.ShapeDtypeStruct((T, K), jnp.float32),
            jax.ShapeDtypeStruct((1, E), jnp.int32),
        ],
        scratch_shapes=[pltpu.VMEM((1, E), jnp.float32)],
    )(x, w_gate)


_PW = A // NW
_KCH = 32
_NCH = _PW // _KCH

@functools.cache
def _sc_mesh():
    return plsc.VectorSubcoreMesh(
        core_axis_name="c", subcore_axis_name="s",
        num_cores=NC, num_subcores=NS)


@functools.cache
def _dispatch_kernel():
    @functools.partial(
        pl.kernel,
        mesh=_sc_mesh(),
        out_type=[
            jax.ShapeDtypeStruct((A_PAD, D), jnp.float32),
            jax.ShapeDtypeStruct((A // _KCH, _KCH), jnp.int32),
        ],
        scratch_types=[
            pltpu.VMEM((_PW,), jnp.int32),
            pltpu.VMEM((_PW,), jnp.int32),
            pltpu.VMEM((E,), jnp.int32),
            pltpu.VMEM((_NCH, _KCH), jnp.int32),
            pltpu.VMEM((_NCH, _KCH), jnp.int32),
            pltpu.VMEM((2, _KCH, D), jnp.float32),
            pltpu.SemaphoreType.DMA((2,)),
            pltpu.SemaphoreType.DMA((2,)),
            pltpu.SemaphoreType.DMA,
        ],
        compiler_params=pltpu.CompilerParams(needs_layout_passes=False),
    )
    def _dispatch_k(x_hbm, ef_hbm, rf_hbm, off_hbm, xs_hbm, dest_hbm,
                    ef_v, rf_v, off_v, tok_v, dst_v, rows_v, sem_g, sem_s,
                    sem_d):
        wid = lax.axis_index("s") * NC + lax.axis_index("c")
        base0 = wid * _PW
        pltpu.sync_copy(off_hbm, off_v)
        pltpu.sync_copy(ef_hbm.at[pl.ds(base0, _PW)], ef_v)
        pltpu.sync_copy(rf_hbm.at[pl.ds(base0, _PW)], rf_v)
        for ci in range(_NCH):
            for s in range(_KCH // LL):
                lo = ci * _KCH + s * LL
                e16 = ef_v[pl.ds(lo, LL)]
                off16 = plsc.load_gather(off_v, [e16])
                dst_v[ci, pl.ds(s * LL, LL)] = off16 + rf_v[pl.ds(lo, LL)]
                t16 = (base0 + lo + lax.iota(jnp.int32, LL)) // K
                tok_v[ci, pl.ds(s * LL, LL)] = t16
            pltpu.make_async_copy(dst_v.at[ci],
                                  dest_hbm.at[wid * _NCH + ci],
                                  sem_d).start()
        pltpu.make_async_copy(x_hbm.at[tok_v.at[0]], rows_v.at[0], sem_g.at[0]
                         ).start()
        for ci in range(_NCH):
            b = ci % 2
            pltpu.make_async_copy(x_hbm.at[tok_v.at[ci]], rows_v.at[b],
                                  sem_g.at[b]).wait()
            if ci + 1 < _NCH:
                if ci >= 1:
                    pltpu.make_async_copy(
                        rows_v.at[1 - b], xs_hbm.at[dst_v.at[ci - 1]],
                        sem_s.at[1 - b]).wait()
                pltpu.make_async_copy(x_hbm.at[tok_v.at[ci + 1]],
                                 rows_v.at[1 - b], sem_g.at[1 - b]).start()
            pltpu.make_async_copy(rows_v.at[b], xs_hbm.at[dst_v.at[ci]],
                             sem_s.at[b]).start()
        pltpu.make_async_copy(rows_v.at[0], xs_hbm.at[dst_v.at[2]],
                              sem_s.at[0]).wait()
        pltpu.make_async_copy(rows_v.at[1], xs_hbm.at[dst_v.at[3]],
                              sem_s.at[1]).wait()
        for ci in range(_NCH):
            pltpu.make_async_copy(dst_v.at[ci],
                                  dest_hbm.at[wid * _NCH + ci],
                                  sem_d).wait()

    return _dispatch_k


def _start_w(w1_ref, w2_ref, e, jj, slot, w1b_ref, w2b_ref, sem_ref):
    pltpu.make_async_copy(
        w1_ref.at[e, :, pl.ds(jj * BF, BF)], w1b_ref.at[slot], sem_ref.at[slot]
    ).start()
    pltpu.make_async_copy(
        w2_ref.at[e, pl.ds(jj * BF, BF), :], w2b_ref.at[slot], sem_ref.at[slot]
    ).start()


def _ffn_body(p_ref, xs_ref, w1_ref, b1_ref, w2_ref, b2_ref, ys_ref,
              w1b_ref, w2b_ref, acc_ref, sem_ref):
    j = pl.program_id(0)
    i = pl.program_id(1)
    e = p_ref[0, i]
    first = p_ref[1, i] == 1
    slot = lax.rem(p_ref[2, i] + j * p_ref[5, 0], 2)

    @pl.when((j == 0) & (i == 0))
    def _():
        _start_w(w1_ref, w2_ref, e, 0, slot, w1b_ref, w2b_ref, sem_ref)

    @pl.when(first)
    def _():
        pltpu.make_async_copy(
            w1_ref.at[0, :, pl.ds(0, BF)], w1b_ref.at[slot], sem_ref.at[slot]
        ).wait()
        pltpu.make_async_copy(
            w2_ref.at[0, pl.ds(0, BF), :], w2b_ref.at[slot], sem_ref.at[slot]
        ).wait()
        ne = p_ref[3, i]
        lr = p_ref[4, i]
        jn = j + lr

        @pl.when(jn < NJ)
        def _():
            _start_w(w1_ref, w2_ref, ne, jn, 1 - slot, w1b_ref, w2b_ref,
                     sem_ref)

    @pl.when(p_ref[6, i] == 1)
    def _():
        h = jnp.dot(xs_ref[...], w1b_ref[slot],
                    preferred_element_type=jnp.float32)
        h = h + b1_ref[0]
        a = jnp.maximum(h, 0.0)
        a = jnp.where(a > THRESHOLD, a, REPLACEMENT)
        p = jnp.dot(a, w2b_ref[slot], preferred_element_type=jnp.float32)
        row = pl.ds(i * BM, BM)

        @pl.when(j == 0)
        def _():
            acc_ref[row, :] = p

        @pl.when(j > 0)
        def _():
            acc_ref[row, :] += p

        @pl.when(j == NJ - 1)
        def _():
            ys_ref[...] = acc_ref[row, :] + b2_ref[0]


def _ffn(pref, xs, W1, b1, W2, b2):
    grid_spec = pltpu.PrefetchScalarGridSpec(
        num_scalar_prefetch=1,
        grid=(NJ, NT),
        in_specs=[
            pl.BlockSpec((BM, D), lambda j, i, p: (i, 0)),
            pl.BlockSpec(memory_space=pltpu.MemorySpace.HBM),
            pl.BlockSpec((1, 1, BF), lambda j, i, p: (p[0, i], 0, j)),
            pl.BlockSpec(memory_space=pltpu.MemorySpace.HBM),
            pl.BlockSpec((1, 1, D), lambda j, i, p: (p[0, i], 0, 0)),
        ],
        out_specs=pl.BlockSpec(
            (BM, D), lambda j, i, p: (jnp.where(j == NJ - 1, i, 0), 0)),
        scratch_shapes=[
            pltpu.VMEM((2, D, BF), jnp.float32),
            pltpu.VMEM((2, BF, D), jnp.float32),
            pltpu.VMEM((A_PAD, D), jnp.float32),
            pltpu.SemaphoreType.DMA((2,)),
        ],
    )
    return pl.pallas_call(
        _ffn_body,
        grid_spec=grid_spec,
        out_shape=jax.ShapeDtypeStruct((A_PAD, D), jnp.float32),
        compiler_params=pltpu.CompilerParams(
            dimension_semantics=("arbitrary", "arbitrary")),
    )(pref, xs, W1, b1.reshape(E, 1, FF), W2, b2.reshape(E, 1, D))


_TPW = T // NW
_TCH = 16
_NCH2 = _TPW // _TCH


@functools.cache
def _combine_kernel():
    @functools.partial(
        pl.kernel,
        mesh=_sc_mesh(),
        out_type=jax.ShapeDtypeStruct((T, D), jnp.float32),
        scratch_types=[
            pltpu.VMEM((_NCH2, K * _TCH), jnp.int32),
            pltpu.VMEM((_TPW, K), jnp.float32),
            pltpu.VMEM((2, K * _TCH, D), jnp.float32),
            pltpu.VMEM((2, _TCH, D), jnp.float32),
            pltpu.SemaphoreType.DMA((2,)),
            pltpu.SemaphoreType.DMA((2,)),
        ],
        compiler_params=pltpu.CompilerParams(needs_layout_passes=False),
    )
    def _combine_k(ys_hbm, dest_hbm, g2_hbm, out_hbm, d_v, g_v, rows_v, o_v,
                   sem_g, sem_o):
        wid = lax.axis_index("s") * NC + lax.axis_index("c")
        t0 = wid * _TPW
        pltpu.sync_copy(dest_hbm.at[pl.ds(wid * _NCH2, _NCH2)], d_v)
        pltpu.sync_copy(g2_hbm.at[pl.ds(t0, _TPW)], g_v)
        pltpu.make_async_copy(ys_hbm.at[d_v.at[0]], rows_v.at[0], sem_g.at[0]
                         ).start()
        for ci in range(_NCH2):
            b = ci % 2
            if ci + 1 < _NCH2:
                pltpu.make_async_copy(ys_hbm.at[d_v.at[ci + 1]],
                                 rows_v.at[1 - b], sem_g.at[1 - b]).start()
            pltpu.make_async_copy(ys_hbm.at[d_v.at[ci]], rows_v.at[b],
                                  sem_g.at[b]).wait()
            if ci >= 2:
                pltpu.make_async_copy(o_v.at[b], out_hbm.at[pl.ds(t0, _TCH)],
                                      sem_o.at[b]).wait()

            def body(t, _):
                tt = jnp.full((LL,), ci * _TCH + t, jnp.int32)
                g0 = plsc.load_gather(g_v, [tt, jnp.zeros((LL,), jnp.int32)])
                g1 = plsc.load_gather(g_v, [tt, jnp.ones((LL,), jnp.int32)])
                for dc in range(D // LL):
                    va = rows_v[b, 2 * t, pl.ds(dc * LL, LL)]
                    vb = rows_v[b, 2 * t + 1, pl.ds(dc * LL, LL)]
                    o_v[b, t, pl.ds(dc * LL, LL)] = va * g0 + vb * g1
                return 0

            lax.fori_loop(0, _TCH, body, 0)
            pltpu.make_async_copy(o_v.at[b],
                             out_hbm.at[pl.ds(t0 + ci * _TCH, _TCH)],
                             sem_o.at[b]).start()
        for b in range(2):
            pltpu.make_async_copy(o_v.at[b], out_hbm.at[pl.ds(t0, _TCH)],
                                  sem_o.at[b]).wait()

    return _combine_k


def kernel(h, w_gate, W1, b1, W2, b2):
    Bb, Ss, Dd = h.shape
    x = h.reshape(T, D)

    e2, r2, g2, cnt = _gating(x, w_gate)

    counts = cnt[0]
    padded = ((counts + BM - 1) // BM) * BM
    ends = jnp.cumsum(padded)
    offsets = (ends - padded).astype(jnp.int32)
    tb = jnp.arange(NT, dtype=jnp.int32) * BM
    eot = jnp.minimum(
        jnp.sum((tb[:, None] >= ends[None, :]).astype(jnp.int32), axis=1),
        E - 1).astype(jnp.int32)
    first = jnp.concatenate([jnp.ones((1,), jnp.int32),
                             (eot[1:] != eot[:-1]).astype(jnp.int32)])
    runidx = jnp.cumsum(first) - 1
    nruns = runidx[-1] + 1
    nxt = jnp.searchsorted(runidx, runidx + 1)
    nexte = eot[jnp.where(nxt >= NT, 0, nxt)]
    lastrun = (runidx == nruns - 1).astype(jnp.int32)
    rpar = jnp.full((NT,), nruns % 2, jnp.int32)
    act = (tb < (offsets + counts)[eot]).astype(jnp.int32)
    pref = jnp.stack([eot, first, runidx % 2, nexte, lastrun, rpar, act]
                     ).astype(jnp.int32)

    xs, dest = _dispatch_kernel()(x, e2.reshape(A), r2.reshape(A), offsets)
    ys = _ffn(pref, xs, W1, b1, W2, b2)
    out = _combine_kernel()(ys, dest, g2)
    return out.reshape(Bb, Ss, Dd)
